# baseline (speedup 1.0000x reference)
import functools

import jax
import jax.numpy as jnp
from jax import lax
from jax.experimental import pallas as pl
from jax.experimental.pallas import tpu as pltpu
from jax.experimental.pallas import tpu_sc as plsc

N = 10000
D = 256
H = 128
E = 160000
NS = 16
CHUNK = 128
CPT = 80
GC = 40
EPT = CPT * CHUNK
E_PER_TILE_REAL = E // NS
PAD_PER_TILE = EPT - E_PER_TILE_REAL
ROWS_PT = 632
N_PAD = NS * ROWS_PT

_mesh = plsc.VectorSubcoreMesh(core_axis_name="c", subcore_axis_name="s")


@functools.partial(
    pl.kernel,
    mesh=_mesh,
    out_type=[
        jax.ShapeDtypeStruct((N_PAD,), jnp.float32),
        jax.ShapeDtypeStruct((N_PAD,), jnp.float32),
    ],
    scratch_types=[
        pltpu.VMEM((CPT, CHUNK), jnp.int32),
        pltpu.VMEM((CHUNK,), jnp.float32),
        pltpu.VMEM((ROWS_PT,), jnp.float32),
        pltpu.VMEM_SHARED((N_PAD,), jnp.float32),
    ],
)
def _deg_kernel(dst0_hbm, dst1_hbm, deg0_hbm, deg1_hbm, dstv, onesv, degv,
                acc):
    c = lax.axis_index("c")
    s = lax.axis_index("s")
    for k in range(CHUNK // 16):
        onesv[pl.ds(k * 16, 16)] = jnp.ones((16,), jnp.float32)
    base = s * ROWS_PT
    for k in range(ROWS_PT // CHUNK):
        pltpu.sync_copy(onesv, acc.at[pl.ds(base + k * CHUNK, CHUNK)])
    rem = ROWS_PT - (ROWS_PT // CHUNK) * CHUNK
    if rem:
        pltpu.sync_copy(
            onesv.at[pl.ds(0, rem)],
            acc.at[pl.ds(base + (ROWS_PT // CHUNK) * CHUNK, rem)],
        )
    plsc.subcore_barrier()

    def _histogram(dst_hbm):
        pltpu.sync_copy(dst_hbm.at[s], dstv)

        def body(j, carry):
            pltpu.sync_copy(onesv, acc.at[dstv.at[j]], add=True)
            return carry

        lax.fori_loop(0, CPT, body, 0)

    @pl.when(c == 0)
    def _():
        _histogram(dst0_hbm)

    @pl.when(c == 1)
    def _():
        _histogram(dst1_hbm)

    plsc.subcore_barrier()

    pltpu.sync_copy(acc.at[pl.ds(base, ROWS_PT)], degv)

    @pl.when(c == 0)
    def _():
        pltpu.sync_copy(degv, deg0_hbm.at[pl.ds(base, ROWS_PT)])

    @pl.when(c == 1)
    def _():
        pltpu.sync_copy(degv, deg1_hbm.at[pl.ds(base, ROWS_PT)])


@functools.partial(
    pl.kernel,
    mesh=_mesh,
    out_type=[
        jax.ShapeDtypeStruct((N_PAD, H), jnp.float32),
        jax.ShapeDtypeStruct((N_PAD, H), jnp.float32),
    ],
    scratch_types=[
        pltpu.VMEM((GC, CHUNK), jnp.int32),
        pltpu.VMEM((GC, CHUNK), jnp.int32),
        pltpu.VMEM((CHUNK, H), jnp.float32),
        pltpu.VMEM((CHUNK, H), jnp.float32),
        pltpu.VMEM_SHARED((N_PAD, H), jnp.float32),
        pltpu.SemaphoreType.DMA,
        pltpu.SemaphoreType.DMA,
        pltpu.SemaphoreType.DMA,
        pltpu.SemaphoreType.DMA,
    ],
)
def _scatter_kernel(hlo_hbm, hhi_hbm, src_hbm, dst_hbm, outlo_hbm, outhi_hbm,
                    srcv, dstv, buf0, buf1, acc, sg0, sg1, ss0, ss1):
    c = lax.axis_index("c")
    s = lax.axis_index("s")
    base = s * ROWS_PT

    def zrow(i, carry):
        for k in range(H // 16):
            buf0[i, pl.ds(k * 16, 16)] = jnp.zeros((16,), jnp.float32)
        return carry

    lax.fori_loop(0, CHUNK, zrow, 0)
    for k in range(ROWS_PT // CHUNK):
        pltpu.sync_copy(buf0, acc.at[pl.ds(base + k * CHUNK, CHUNK)])
    rem = ROWS_PT - (ROWS_PT // CHUNK) * CHUNK
    if rem:
        pltpu.sync_copy(
            buf0.at[pl.ds(0, rem)],
            acc.at[pl.ds(base + (ROWS_PT // CHUNK) * CHUNK, rem)],
        )
    plsc.subcore_barrier()

    def run(h_ref):
        def gather(j, buf, sem):
            return pltpu.async_copy(h_ref.at[srcv.at[j]], buf, sem)

        def scat(j, buf, sem):
            return pltpu.async_copy(buf, acc.at[dstv.at[j]], sem, add=True)

        def wait_g(j, buf, sem):
            pltpu.make_async_copy(h_ref.at[srcv.at[j]], buf, sem).wait()

        def wait_s(j, buf, sem):
            pltpu.make_async_copy(buf, acc.at[dstv.at[j]], sem).wait()

        def group(g, carry):
            pltpu.sync_copy(src_hbm.at[s, pl.ds(g * GC, GC)], srcv)
            pltpu.sync_copy(dst_hbm.at[s, pl.ds(g * GC, GC)], dstv)
            gather(0, buf0, sg0)

            def body(i, carry2):
                j0 = 2 * i
                wait_g(j0, buf0, sg0)
                scat(j0, buf0, ss0)

                @pl.when(i > 0)
                def _():
                    wait_s(j0 - 1, buf1, ss1)

                gather(j0 + 1, buf1, sg1)
                wait_g(j0 + 1, buf1, sg1)
                scat(j0 + 1, buf1, ss1)
                wait_s(j0, buf0, ss0)

                @pl.when(j0 + 2 < GC)
                def _():
                    gather(j0 + 2, buf0, sg0)

                return carry2

            lax.fori_loop(0, GC // 2, body, 0)
            wait_s(GC - 1, buf1, ss1)
            return carry

        lax.fori_loop(0, CPT // GC, group, 0)

    @pl.when(c == 0)
    def _():
        run(hlo_hbm)

    @pl.when(c == 1)
    def _():
        run(hhi_hbm)

    plsc.subcore_barrier()

    def drain(out_hbm):
        for k in range(ROWS_PT // CHUNK):
            pltpu.sync_copy(acc.at[pl.ds(base + k * CHUNK, CHUNK)], buf0)
            pltpu.sync_copy(buf0, out_hbm.at[pl.ds(base + k * CHUNK, CHUNK)])
        rem2 = ROWS_PT - (ROWS_PT // CHUNK) * CHUNK
        if rem2:
            off = base + (ROWS_PT // CHUNK) * CHUNK
            pltpu.sync_copy(acc.at[pl.ds(off, rem2)],
                            buf0.at[pl.ds(0, rem2)])
            pltpu.sync_copy(buf0.at[pl.ds(0, rem2)],
                            out_hbm.at[pl.ds(off, rem2)])

    @pl.when(c == 0)
    def _():
        drain(outlo_hbm)

    @pl.when(c == 1)
    def _():
        drain(outhi_hbm)


_RB = 400
_GRID = N // _RB


def _b0_body(x_ref, w_ref, deg_ref, lo_ref, hi_ref):
    a = lax.rsqrt(deg_ref[...])
    h = jnp.dot(x_ref[...], w_ref[...], preferred_element_type=jnp.float32) * a
    lo_ref[...] = h[:, :H]
    hi_ref[...] = h[:, H:]


def _mm_scale(x, w, deg):
    return pl.pallas_call(
        _b0_body,
        grid=(_GRID,),
        in_specs=[
            pl.BlockSpec((_RB, D), lambda i: (i, 0)),
            pl.BlockSpec((D, D), lambda i: (0, 0)),
            pl.BlockSpec((_RB, 1), lambda i: (i, 0)),
        ],
        out_specs=[
            pl.BlockSpec((_RB, H), lambda i: (i, 0)),
            pl.BlockSpec((_RB, H), lambda i: (i, 0)),
        ],
        out_shape=[
            jax.ShapeDtypeStruct((N, H), jnp.float32),
            jax.ShapeDtypeStruct((N, H), jnp.float32),
        ],
    )(x, w, deg)


def _b1_body(slo_ref, shi_ref, hlo_ref, hhi_ref, deg0_ref, b_ref, w_ref,
             deg1_ref, lo_ref, hi_ref):
    a0 = lax.rsqrt(deg0_ref[...])
    sfull = jnp.concatenate([slo_ref[...], shi_ref[...]], axis=1)
    hfull = jnp.concatenate([hlo_ref[...], hhi_ref[...]], axis=1)
    z = jnp.maximum(a0 * (sfull + hfull) + b_ref[...], 0.0)
    h1 = jnp.dot(z, w_ref[...], preferred_element_type=jnp.float32)
    h1 = h1 * lax.rsqrt(deg1_ref[...])
    lo_ref[...] = h1[:, :H]
    hi_ref[...] = h1[:, H:]


def _fused_mid(slo, shi, hlo, hhi, deg0, b0, w1, deg1):
    return pl.pallas_call(
        _b1_body,
        grid=(_GRID,),
        in_specs=[
            pl.BlockSpec((_RB, H), lambda i: (i, 0)),
            pl.BlockSpec((_RB, H), lambda i: (i, 0)),
            pl.BlockSpec((_RB, H), lambda i: (i, 0)),
            pl.BlockSpec((_RB, H), lambda i: (i, 0)),
            pl.BlockSpec((_RB, 1), lambda i: (i, 0)),
            pl.BlockSpec((1, D), lambda i: (0, 0)),
            pl.BlockSpec((D, D), lambda i: (0, 0)),
            pl.BlockSpec((_RB, 1), lambda i: (i, 0)),
        ],
        out_specs=[
            pl.BlockSpec((_RB, H), lambda i: (i, 0)),
            pl.BlockSpec((_RB, H), lambda i: (i, 0)),
        ],
        out_shape=[
            jax.ShapeDtypeStruct((N, H), jnp.float32),
            jax.ShapeDtypeStruct((N, H), jnp.float32),
        ],
    )(slo, shi, hlo, hhi, deg0, b0, w1, deg1)


def _b2_body(slo_ref, shi_ref, hlo_ref, hhi_ref, deg_ref, b_ref, out_ref):
    a = lax.rsqrt(deg_ref[...])
    sfull = jnp.concatenate([slo_ref[...], shi_ref[...]], axis=1)
    hfull = jnp.concatenate([hlo_ref[...], hhi_ref[...]], axis=1)
    out_ref[...] = a * (sfull + hfull) + b_ref[...]


def _final(slo, shi, hlo, hhi, deg, b):
    return pl.pallas_call(
        _b2_body,
        grid=(_GRID,),
        in_specs=[
            pl.BlockSpec((_RB, H), lambda i: (i, 0)),
            pl.BlockSpec((_RB, H), lambda i: (i, 0)),
            pl.BlockSpec((_RB, H), lambda i: (i, 0)),
            pl.BlockSpec((_RB, H), lambda i: (i, 0)),
            pl.BlockSpec((_RB, 1), lambda i: (i, 0)),
            pl.BlockSpec((1, D), lambda i: (0, 0)),
        ],
        out_specs=pl.BlockSpec((_RB, D), lambda i: (i, 0)),
        out_shape=jax.ShapeDtypeStruct((N, D), jnp.float32),
    )(slo, shi, hlo, hhi, deg, b)


def _prep_edges(src, dst):
    pad_src = (jnp.arange(NS * PAD_PER_TILE, dtype=jnp.int32) * 37) % N
    pad_dst = jnp.full((NS * PAD_PER_TILE,), N, jnp.int32)
    src_t = jnp.concatenate(
        [src.reshape(NS, E_PER_TILE_REAL), pad_src.reshape(NS, PAD_PER_TILE)],
        axis=1).reshape(NS, CPT, CHUNK)
    dst_t = jnp.concatenate(
        [dst.reshape(NS, E_PER_TILE_REAL), pad_dst.reshape(NS, PAD_PER_TILE)],
        axis=1).reshape(NS, CPT, CHUNK)
    return src_t, dst_t


def kernel(x, edge_index_list, W0, b0, W1, b1):
    e = edge_index_list.astype(jnp.int32)
    src0, dst0 = _prep_edges(e[0, 0], e[0, 1])
    src1, dst1 = _prep_edges(e[1, 0], e[1, 1])

    deg0, deg1 = _deg_kernel(dst0, dst1)
    deg0 = deg0.reshape(N_PAD, 1)
    deg1 = deg1.reshape(N_PAD, 1)

    h0lo, h0hi = _mm_scale(x, W0, deg0)
    s0lo, s0hi = _scatter_kernel(h0lo, h0hi, src0, dst0)
    h1lo, h1hi = _fused_mid(s0lo, s0hi, h0lo, h0hi, deg0,
                            b0.reshape(1, D), W1, deg1)
    s1lo, s1hi = _scatter_kernel(h1lo, h1hi, src1, dst1)
    return _final(s1lo, s1hi, h1lo, h1hi, deg1, b1.reshape(1, D))


# speedup vs baseline: 1.1267x; 1.1267x over previous
import functools

import jax
import jax.numpy as jnp
from jax import lax
from jax.experimental import pallas as pl
from jax.experimental.pallas import tpu as pltpu
from jax.experimental.pallas import tpu_sc as plsc

N = 10000
D = 256
H = 128
E = 160000
NS = 16
CHUNK = 128
CPT = 80
GC = 40
EPT = CPT * CHUNK
E_PER_TILE_REAL = E // NS
PAD_PER_TILE = EPT - E_PER_TILE_REAL
ROWS_PT = 632
N_PAD = NS * ROWS_PT

_mesh = plsc.VectorSubcoreMesh(core_axis_name="c", subcore_axis_name="s")


@functools.partial(
    pl.kernel,
    mesh=_mesh,
    out_type=[
        jax.ShapeDtypeStruct((N_PAD,), jnp.float32),
        jax.ShapeDtypeStruct((N_PAD,), jnp.float32),
    ],
    scratch_types=[
        pltpu.VMEM((CPT, CHUNK), jnp.int32),
        pltpu.VMEM((CHUNK,), jnp.float32),
        pltpu.VMEM((ROWS_PT,), jnp.float32),
        pltpu.VMEM_SHARED((N_PAD,), jnp.float32),
    ],
)
def _deg_kernel(dst0_hbm, dst1_hbm, deg0_hbm, deg1_hbm, dstv, onesv, degv,
                acc):
    c = lax.axis_index("c")
    s = lax.axis_index("s")
    for k in range(CHUNK // 16):
        onesv[pl.ds(k * 16, 16)] = jnp.ones((16,), jnp.float32)
    base = s * ROWS_PT
    for k in range(ROWS_PT // CHUNK):
        pltpu.sync_copy(onesv, acc.at[pl.ds(base + k * CHUNK, CHUNK)])
    rem = ROWS_PT - (ROWS_PT // CHUNK) * CHUNK
    if rem:
        pltpu.sync_copy(
            onesv.at[pl.ds(0, rem)],
            acc.at[pl.ds(base + (ROWS_PT // CHUNK) * CHUNK, rem)],
        )
    plsc.subcore_barrier()

    def _histogram(dst_hbm):
        pltpu.sync_copy(dst_hbm.at[s], dstv)

        def body(j, carry):
            pltpu.sync_copy(onesv, acc.at[dstv.at[j]], add=True)
            return carry

        lax.fori_loop(0, CPT, body, 0)

    @pl.when(c == 0)
    def _():
        _histogram(dst0_hbm)

    @pl.when(c == 1)
    def _():
        _histogram(dst1_hbm)

    plsc.subcore_barrier()

    pltpu.sync_copy(acc.at[pl.ds(base, ROWS_PT)], degv)

    @pl.when(c == 0)
    def _():
        pltpu.sync_copy(degv, deg0_hbm.at[pl.ds(base, ROWS_PT)])

    @pl.when(c == 1)
    def _():
        pltpu.sync_copy(degv, deg1_hbm.at[pl.ds(base, ROWS_PT)])


@functools.partial(
    pl.kernel,
    mesh=_mesh,
    out_type=[
        jax.ShapeDtypeStruct((N_PAD, H), jnp.float32),
        jax.ShapeDtypeStruct((N_PAD, H), jnp.float32),
    ],
    scratch_types=[
        pltpu.VMEM((GC, CHUNK), jnp.int32),
        pltpu.VMEM((GC, CHUNK), jnp.int32),
        pltpu.VMEM((CHUNK, H), jnp.float32),
        pltpu.VMEM((CHUNK, H), jnp.float32),
        pltpu.VMEM_SHARED((N_PAD, H), jnp.float32),
        pltpu.SemaphoreType.DMA,
        pltpu.SemaphoreType.DMA,
        pltpu.SemaphoreType.DMA,
        pltpu.SemaphoreType.DMA,
    ],
)
def _scatter_kernel(hlo_hbm, hhi_hbm, src_hbm, dst_hbm, outlo_hbm, outhi_hbm,
                    srcv, dstv, buf0, buf1, acc, sg0, sg1, ss0, ss1):
    c = lax.axis_index("c")
    s = lax.axis_index("s")
    base = s * ROWS_PT

    def zrow(i, carry):
        for k in range(H // 16):
            buf0[i, pl.ds(k * 16, 16)] = jnp.zeros((16,), jnp.float32)
        return carry

    lax.fori_loop(0, CHUNK, zrow, 0)
    for k in range(ROWS_PT // CHUNK):
        pltpu.sync_copy(buf0, acc.at[pl.ds(base + k * CHUNK, CHUNK)])
    rem = ROWS_PT - (ROWS_PT // CHUNK) * CHUNK
    if rem:
        pltpu.sync_copy(
            buf0.at[pl.ds(0, rem)],
            acc.at[pl.ds(base + (ROWS_PT // CHUNK) * CHUNK, rem)],
        )
    plsc.subcore_barrier()

    def run(h_ref):
        def gather(j, buf, sem):
            return pltpu.async_copy(h_ref.at[srcv.at[j]], buf, sem)

        def scat(j, buf, sem):
            return pltpu.async_copy(buf, acc.at[dstv.at[j]], sem, add=True)

        def wait_g(j, buf, sem):
            pltpu.make_async_copy(h_ref.at[srcv.at[j]], buf, sem).wait()

        def wait_s(j, buf, sem):
            pltpu.make_async_copy(buf, acc.at[dstv.at[j]], sem).wait()

        def group(g, carry):
            pltpu.sync_copy(src_hbm.at[s, pl.ds(g * GC, GC)], srcv)
            pltpu.sync_copy(dst_hbm.at[s, pl.ds(g * GC, GC)], dstv)
            gather(0, buf0, sg0)

            def body(i, carry2):
                j0 = 2 * i
                gather(j0 + 1, buf1, sg1)
                wait_g(j0, buf0, sg0)
                pltpu.sync_copy(buf0, acc.at[dstv.at[j0]], add=True)

                @pl.when(j0 + 2 < GC)
                def _():
                    gather(j0 + 2, buf0, sg0)

                wait_g(j0 + 1, buf1, sg1)
                pltpu.sync_copy(buf1, acc.at[dstv.at[j0 + 1]], add=True)
                return carry2

            lax.fori_loop(0, GC // 2, body, 0)
            return carry

        lax.fori_loop(0, CPT // GC, group, 0)

    @pl.when(c == 0)
    def _():
        run(hlo_hbm)

    @pl.when(c == 1)
    def _():
        run(hhi_hbm)

    plsc.subcore_barrier()

    def drain(out_hbm):
        for k in range(ROWS_PT // CHUNK):
            pltpu.sync_copy(acc.at[pl.ds(base + k * CHUNK, CHUNK)], buf0)
            pltpu.sync_copy(buf0, out_hbm.at[pl.ds(base + k * CHUNK, CHUNK)])
        rem2 = ROWS_PT - (ROWS_PT // CHUNK) * CHUNK
        if rem2:
            off = base + (ROWS_PT // CHUNK) * CHUNK
            pltpu.sync_copy(acc.at[pl.ds(off, rem2)],
                            buf0.at[pl.ds(0, rem2)])
            pltpu.sync_copy(buf0.at[pl.ds(0, rem2)],
                            out_hbm.at[pl.ds(off, rem2)])

    @pl.when(c == 0)
    def _():
        drain(outlo_hbm)

    @pl.when(c == 1)
    def _():
        drain(outhi_hbm)


_RB = 400
_GRID = N // _RB


def _b0_body(x_ref, w_ref, deg_ref, lo_ref, hi_ref):
    a = lax.rsqrt(deg_ref[...])
    h = jnp.dot(x_ref[...], w_ref[...], preferred_element_type=jnp.float32) * a
    lo_ref[...] = h[:, :H]
    hi_ref[...] = h[:, H:]


def _mm_scale(x, w, deg):
    return pl.pallas_call(
        _b0_body,
        grid=(_GRID,),
        in_specs=[
            pl.BlockSpec((_RB, D), lambda i: (i, 0)),
            pl.BlockSpec((D, D), lambda i: (0, 0)),
            pl.BlockSpec((_RB, 1), lambda i: (i, 0)),
        ],
        out_specs=[
            pl.BlockSpec((_RB, H), lambda i: (i, 0)),
            pl.BlockSpec((_RB, H), lambda i: (i, 0)),
        ],
        out_shape=[
            jax.ShapeDtypeStruct((N, H), jnp.float32),
            jax.ShapeDtypeStruct((N, H), jnp.float32),
        ],
    )(x, w, deg)


def _b1_body(slo_ref, shi_ref, hlo_ref, hhi_ref, deg0_ref, b_ref, w_ref,
             deg1_ref, lo_ref, hi_ref):
    a0 = lax.rsqrt(deg0_ref[...])
    sfull = jnp.concatenate([slo_ref[...], shi_ref[...]], axis=1)
    hfull = jnp.concatenate([hlo_ref[...], hhi_ref[...]], axis=1)
    z = jnp.maximum(a0 * (sfull + hfull) + b_ref[...], 0.0)
    h1 = jnp.dot(z, w_ref[...], preferred_element_type=jnp.float32)
    h1 = h1 * lax.rsqrt(deg1_ref[...])
    lo_ref[...] = h1[:, :H]
    hi_ref[...] = h1[:, H:]


def _fused_mid(slo, shi, hlo, hhi, deg0, b0, w1, deg1):
    return pl.pallas_call(
        _b1_body,
        grid=(_GRID,),
        in_specs=[
            pl.BlockSpec((_RB, H), lambda i: (i, 0)),
            pl.BlockSpec((_RB, H), lambda i: (i, 0)),
            pl.BlockSpec((_RB, H), lambda i: (i, 0)),
            pl.BlockSpec((_RB, H), lambda i: (i, 0)),
            pl.BlockSpec((_RB, 1), lambda i: (i, 0)),
            pl.BlockSpec((1, D), lambda i: (0, 0)),
            pl.BlockSpec((D, D), lambda i: (0, 0)),
            pl.BlockSpec((_RB, 1), lambda i: (i, 0)),
        ],
        out_specs=[
            pl.BlockSpec((_RB, H), lambda i: (i, 0)),
            pl.BlockSpec((_RB, H), lambda i: (i, 0)),
        ],
        out_shape=[
            jax.ShapeDtypeStruct((N, H), jnp.float32),
            jax.ShapeDtypeStruct((N, H), jnp.float32),
        ],
    )(slo, shi, hlo, hhi, deg0, b0, w1, deg1)


def _b2_body(slo_ref, shi_ref, hlo_ref, hhi_ref, deg_ref, b_ref, out_ref):
    a = lax.rsqrt(deg_ref[...])
    sfull = jnp.concatenate([slo_ref[...], shi_ref[...]], axis=1)
    hfull = jnp.concatenate([hlo_ref[...], hhi_ref[...]], axis=1)
    out_ref[...] = a * (sfull + hfull) + b_ref[...]


def _final(slo, shi, hlo, hhi, deg, b):
    return pl.pallas_call(
        _b2_body,
        grid=(_GRID,),
        in_specs=[
            pl.BlockSpec((_RB, H), lambda i: (i, 0)),
            pl.BlockSpec((_RB, H), lambda i: (i, 0)),
            pl.BlockSpec((_RB, H), lambda i: (i, 0)),
            pl.BlockSpec((_RB, H), lambda i: (i, 0)),
            pl.BlockSpec((_RB, 1), lambda i: (i, 0)),
            pl.BlockSpec((1, D), lambda i: (0, 0)),
        ],
        out_specs=pl.BlockSpec((_RB, D), lambda i: (i, 0)),
        out_shape=jax.ShapeDtypeStruct((N, D), jnp.float32),
    )(slo, shi, hlo, hhi, deg, b)


def _prep_edges(src, dst):
    pad_src = (jnp.arange(NS * PAD_PER_TILE, dtype=jnp.int32) * 37) % N
    pad_dst = jnp.full((NS * PAD_PER_TILE,), N, jnp.int32)
    src_t = jnp.concatenate(
        [src.reshape(NS, E_PER_TILE_REAL), pad_src.reshape(NS, PAD_PER_TILE)],
        axis=1).reshape(NS, CPT, CHUNK)
    dst_t = jnp.concatenate(
        [dst.reshape(NS, E_PER_TILE_REAL), pad_dst.reshape(NS, PAD_PER_TILE)],
        axis=1).reshape(NS, CPT, CHUNK)
    return src_t, dst_t


def kernel(x, edge_index_list, W0, b0, W1, b1):
    e = edge_index_list.astype(jnp.int32)
    src0, dst0 = _prep_edges(e[0, 0], e[0, 1])
    src1, dst1 = _prep_edges(e[1, 0], e[1, 1])

    deg0, deg1 = _deg_kernel(dst0, dst1)
    deg0 = deg0.reshape(N_PAD, 1)
    deg1 = deg1.reshape(N_PAD, 1)

    h0lo, h0hi = _mm_scale(x, W0, deg0)
    s0lo, s0hi = _scatter_kernel(h0lo, h0hi, src0, dst0)
    h1lo, h1hi = _fused_mid(s0lo, s0hi, h0lo, h0hi, deg0,
                            b0.reshape(1, D), W1, deg1)
    s1lo, s1hi = _scatter_kernel(h1lo, h1hi, src1, dst1)
    return _final(s1lo, s1hi, h1lo, h1hi, deg1, b1.reshape(1, D))


# speedup vs baseline: 1.1499x; 1.0205x over previous
import functools

import jax
import jax.numpy as jnp
from jax import lax
from jax.experimental import pallas as pl
from jax.experimental.pallas import tpu as pltpu
from jax.experimental.pallas import tpu_sc as plsc

N = 10000
D = 256
H = 128
E = 160000
NS = 16
CHUNK = 128
CPT = 80
GC = 40
EPT = CPT * CHUNK
E_PER_TILE_REAL = E // NS
PAD_PER_TILE = EPT - E_PER_TILE_REAL
ROWS_PT = 632
N_PAD = NS * ROWS_PT

_mesh = plsc.VectorSubcoreMesh(core_axis_name="c", subcore_axis_name="s")


@functools.partial(
    pl.kernel,
    mesh=_mesh,
    out_type=[
        jax.ShapeDtypeStruct((N_PAD,), jnp.float32),
        jax.ShapeDtypeStruct((N_PAD,), jnp.float32),
    ],
    scratch_types=[
        pltpu.VMEM((CPT, CHUNK), jnp.int32),
        pltpu.VMEM((CPT, CHUNK), jnp.float32),
        pltpu.VMEM((ROWS_PT,), jnp.float32),
        pltpu.VMEM_SHARED((N_PAD,), jnp.float32),
    ],
)
def _deg_kernel(dst0_hbm, dst1_hbm, deg0_hbm, deg1_hbm, dstv, onesv, degv,
                acc):
    c = lax.axis_index("c")
    s = lax.axis_index("s")

    def orow(i, carry):
        for k in range(CHUNK // 16):
            onesv[i, pl.ds(k * 16, 16)] = jnp.ones((16,), jnp.float32)
        return carry

    lax.fori_loop(0, CPT, orow, 0)
    base = s * ROWS_PT
    for k in range(ROWS_PT // CHUNK):
        pltpu.sync_copy(onesv.at[0], acc.at[pl.ds(base + k * CHUNK, CHUNK)])
    rem = ROWS_PT - (ROWS_PT // CHUNK) * CHUNK
    if rem:
        pltpu.sync_copy(
            onesv.at[0, pl.ds(0, rem)],
            acc.at[pl.ds(base + (ROWS_PT // CHUNK) * CHUNK, rem)],
        )
    plsc.subcore_barrier()

    def _histogram(dst_hbm):
        pltpu.sync_copy(dst_hbm.at[s], dstv)

        def body(j, carry):
            pltpu.sync_copy(onesv.at[j], acc.at[dstv.at[j]], add=True)
            return carry

        lax.fori_loop(0, CPT, body, 0)

    @pl.when(c == 0)
    def _():
        _histogram(dst0_hbm)

    @pl.when(c == 1)
    def _():
        _histogram(dst1_hbm)

    plsc.subcore_barrier()

    pltpu.sync_copy(acc.at[pl.ds(base, ROWS_PT)], degv)

    @pl.when(c == 0)
    def _():
        pltpu.sync_copy(degv, deg0_hbm.at[pl.ds(base, ROWS_PT)])

    @pl.when(c == 1)
    def _():
        pltpu.sync_copy(degv, deg1_hbm.at[pl.ds(base, ROWS_PT)])


@functools.partial(
    pl.kernel,
    mesh=_mesh,
    out_type=[
        jax.ShapeDtypeStruct((N_PAD, H), jnp.float32),
        jax.ShapeDtypeStruct((N_PAD, H), jnp.float32),
    ],
    scratch_types=[
        pltpu.VMEM((GC, CHUNK), jnp.int32),
        pltpu.VMEM((GC, CHUNK), jnp.int32),
        pltpu.VMEM((CHUNK, H), jnp.float32),
        pltpu.VMEM((CHUNK, H), jnp.float32),
        pltpu.VMEM_SHARED((N_PAD, H), jnp.float32),
        pltpu.SemaphoreType.DMA,
        pltpu.SemaphoreType.DMA,
        pltpu.SemaphoreType.DMA,
        pltpu.SemaphoreType.DMA,
    ],
)
def _scatter_kernel(hlo_hbm, hhi_hbm, src_hbm, dst_hbm, outlo_hbm, outhi_hbm,
                    srcv, dstv, buf0, buf1, acc, sg0, sg1, ss0, ss1):
    c = lax.axis_index("c")
    s = lax.axis_index("s")
    base = s * ROWS_PT

    def zrow(i, carry):
        for k in range(H // 16):
            buf0[i, pl.ds(k * 16, 16)] = jnp.zeros((16,), jnp.float32)
        return carry

    pltpu.async_copy(src_hbm.at[s, pl.ds(0, GC)], srcv, sg0)
    pltpu.async_copy(dst_hbm.at[s, pl.ds(0, GC)], dstv, sg1)
    lax.fori_loop(0, CHUNK, zrow, 0)
    for k in range(ROWS_PT // CHUNK):
        pltpu.sync_copy(buf0, acc.at[pl.ds(base + k * CHUNK, CHUNK)])
    rem = ROWS_PT - (ROWS_PT // CHUNK) * CHUNK
    if rem:
        pltpu.sync_copy(
            buf0.at[pl.ds(0, rem)],
            acc.at[pl.ds(base + (ROWS_PT // CHUNK) * CHUNK, rem)],
        )
    pltpu.make_async_copy(src_hbm.at[s, pl.ds(0, GC)], srcv, sg0).wait()
    pltpu.make_async_copy(dst_hbm.at[s, pl.ds(0, GC)], dstv, sg1).wait()
    plsc.subcore_barrier()

    def run(h_ref):
        def gather(j, buf, sem):
            return pltpu.async_copy(h_ref.at[srcv.at[j]], buf, sem)

        def scat(j, buf, sem):
            return pltpu.async_copy(buf, acc.at[dstv.at[j]], sem, add=True)

        def wait_g(j, buf, sem):
            pltpu.make_async_copy(h_ref.at[srcv.at[j]], buf, sem).wait()

        def wait_s(j, buf, sem):
            pltpu.make_async_copy(buf, acc.at[dstv.at[j]], sem).wait()

        def group(g, carry):
            @pl.when(g > 0)
            def _():
                pltpu.sync_copy(src_hbm.at[s, pl.ds(g * GC, GC)], srcv)
                pltpu.sync_copy(dst_hbm.at[s, pl.ds(g * GC, GC)], dstv)

            gather(0, buf0, sg0)

            def body(i, carry2):
                j0 = 2 * i
                gather(j0 + 1, buf1, sg1)
                wait_g(j0, buf0, sg0)
                pltpu.sync_copy(buf0, acc.at[dstv.at[j0]], add=True)

                @pl.when(j0 + 2 < GC)
                def _():
                    gather(j0 + 2, buf0, sg0)

                wait_g(j0 + 1, buf1, sg1)
                pltpu.sync_copy(buf1, acc.at[dstv.at[j0 + 1]], add=True)
                return carry2

            lax.fori_loop(0, GC // 2, body, 0)
            return carry

        lax.fori_loop(0, CPT // GC, group, 0)

    @pl.when(c == 0)
    def _():
        run(hlo_hbm)

    @pl.when(c == 1)
    def _():
        run(hhi_hbm)

    plsc.subcore_barrier()

    def drain(out_hbm):
        hops = [(k * CHUNK, CHUNK) for k in range(ROWS_PT // CHUNK)]
        rem2 = ROWS_PT - (ROWS_PT // CHUNK) * CHUNK
        if rem2:
            hops.append(((ROWS_PT // CHUNK) * CHUNK, rem2))
        pend = [None, None]
        for k, (off, sz) in enumerate(hops):
            buf = (buf0 if k % 2 == 0 else buf1).at[pl.ds(0, sz)]
            sem = sg0 if k % 2 == 0 else sg1
            if pend[k % 2] is not None:
                pend[k % 2].wait()
            pltpu.sync_copy(acc.at[pl.ds(base + off, sz)], buf)
            pltpu.async_copy(buf, out_hbm.at[pl.ds(base + off, sz)], sem)
            pend[k % 2] = pltpu.make_async_copy(
                buf, out_hbm.at[pl.ds(base + off, sz)], sem)
        for p in pend:
            if p is not None:
                p.wait()

    @pl.when(c == 0)
    def _():
        drain(outlo_hbm)

    @pl.when(c == 1)
    def _():
        drain(outhi_hbm)


_RB = 400
_GRID = N // _RB


def _b0_body(x_ref, w_ref, deg_ref, lo_ref, hi_ref):
    a = lax.rsqrt(deg_ref[...])
    h = jnp.dot(x_ref[...], w_ref[...], preferred_element_type=jnp.float32) * a
    lo_ref[...] = h[:, :H]
    hi_ref[...] = h[:, H:]


def _mm_scale(x, w, deg):
    return pl.pallas_call(
        _b0_body,
        grid=(_GRID,),
        in_specs=[
            pl.BlockSpec((_RB, D), lambda i: (i, 0)),
            pl.BlockSpec((D, D), lambda i: (0, 0)),
            pl.BlockSpec((_RB, 1), lambda i: (i, 0)),
        ],
        out_specs=[
            pl.BlockSpec((_RB, H), lambda i: (i, 0)),
            pl.BlockSpec((_RB, H), lambda i: (i, 0)),
        ],
        out_shape=[
            jax.ShapeDtypeStruct((N, H), jnp.float32),
            jax.ShapeDtypeStruct((N, H), jnp.float32),
        ],
    )(x, w, deg)


def _b1_body(slo_ref, shi_ref, hlo_ref, hhi_ref, deg0_ref, b_ref, w_ref,
             deg1_ref, lo_ref, hi_ref):
    a0 = lax.rsqrt(deg0_ref[...])
    sfull = jnp.concatenate([slo_ref[...], shi_ref[...]], axis=1)
    hfull = jnp.concatenate([hlo_ref[...], hhi_ref[...]], axis=1)
    z = jnp.maximum(a0 * (sfull + hfull) + b_ref[...], 0.0)
    h1 = jnp.dot(z, w_ref[...], preferred_element_type=jnp.float32)
    h1 = h1 * lax.rsqrt(deg1_ref[...])
    lo_ref[...] = h1[:, :H]
    hi_ref[...] = h1[:, H:]


def _fused_mid(slo, shi, hlo, hhi, deg0, b0, w1, deg1):
    return pl.pallas_call(
        _b1_body,
        grid=(_GRID,),
        in_specs=[
            pl.BlockSpec((_RB, H), lambda i: (i, 0)),
            pl.BlockSpec((_RB, H), lambda i: (i, 0)),
            pl.BlockSpec((_RB, H), lambda i: (i, 0)),
            pl.BlockSpec((_RB, H), lambda i: (i, 0)),
            pl.BlockSpec((_RB, 1), lambda i: (i, 0)),
            pl.BlockSpec((1, D), lambda i: (0, 0)),
            pl.BlockSpec((D, D), lambda i: (0, 0)),
            pl.BlockSpec((_RB, 1), lambda i: (i, 0)),
        ],
        out_specs=[
            pl.BlockSpec((_RB, H), lambda i: (i, 0)),
            pl.BlockSpec((_RB, H), lambda i: (i, 0)),
        ],
        out_shape=[
            jax.ShapeDtypeStruct((N, H), jnp.float32),
            jax.ShapeDtypeStruct((N, H), jnp.float32),
        ],
    )(slo, shi, hlo, hhi, deg0, b0, w1, deg1)


def _b2_body(slo_ref, shi_ref, hlo_ref, hhi_ref, deg_ref, b_ref, out_ref):
    a = lax.rsqrt(deg_ref[...])
    sfull = jnp.concatenate([slo_ref[...], shi_ref[...]], axis=1)
    hfull = jnp.concatenate([hlo_ref[...], hhi_ref[...]], axis=1)
    out_ref[...] = a * (sfull + hfull) + b_ref[...]


def _final(slo, shi, hlo, hhi, deg, b):
    return pl.pallas_call(
        _b2_body,
        grid=(_GRID,),
        in_specs=[
            pl.BlockSpec((_RB, H), lambda i: (i, 0)),
            pl.BlockSpec((_RB, H), lambda i: (i, 0)),
            pl.BlockSpec((_RB, H), lambda i: (i, 0)),
            pl.BlockSpec((_RB, H), lambda i: (i, 0)),
            pl.BlockSpec((_RB, 1), lambda i: (i, 0)),
            pl.BlockSpec((1, D), lambda i: (0, 0)),
        ],
        out_specs=pl.BlockSpec((_RB, D), lambda i: (i, 0)),
        out_shape=jax.ShapeDtypeStruct((N, D), jnp.float32),
    )(slo, shi, hlo, hhi, deg, b)


def _prep_edges(src, dst):
    pad_src = (jnp.arange(NS * PAD_PER_TILE, dtype=jnp.int32) * 37) % N
    pad_dst = jnp.full((NS * PAD_PER_TILE,), N, jnp.int32)
    src_t = jnp.concatenate(
        [src.reshape(NS, E_PER_TILE_REAL), pad_src.reshape(NS, PAD_PER_TILE)],
        axis=1).reshape(NS, CPT, CHUNK)
    dst_t = jnp.concatenate(
        [dst.reshape(NS, E_PER_TILE_REAL), pad_dst.reshape(NS, PAD_PER_TILE)],
        axis=1).reshape(NS, CPT, CHUNK)
    return src_t, dst_t


def kernel(x, edge_index_list, W0, b0, W1, b1):
    e = edge_index_list.astype(jnp.int32)
    src0, dst0 = _prep_edges(e[0, 0], e[0, 1])
    src1, dst1 = _prep_edges(e[1, 0], e[1, 1])

    deg0, deg1 = _deg_kernel(dst0, dst1)
    deg0 = deg0.reshape(N_PAD, 1)
    deg1 = deg1.reshape(N_PAD, 1)

    h0lo, h0hi = _mm_scale(x, W0, deg0)
    s0lo, s0hi = _scatter_kernel(h0lo, h0hi, src0, dst0)
    h1lo, h1hi = _fused_mid(s0lo, s0hi, h0lo, h0hi, deg0,
                            b0.reshape(1, D), W1, deg1)
    s1lo, s1hi = _scatter_kernel(h1lo, h1hi, src1, dst1)
    return _final(s1lo, s1hi, h1lo, h1hi, deg1, b1.reshape(1, D))


# speedup vs baseline: 1.2610x; 1.0966x over previous
import functools

import jax
import jax.numpy as jnp
from jax import lax
from jax.experimental import pallas as pl
from jax.experimental.pallas import tpu as pltpu
from jax.experimental.pallas import tpu_sc as plsc

N = 10000
D = 256
H = 128
E = 160000
NS = 16
CHUNK = 128
CPT = 80
GC = 40
EPT = CPT * CHUNK
E_PER_TILE_REAL = E // NS
PAD_PER_TILE = EPT - E_PER_TILE_REAL
ROWS_PT = 632
N_PAD = NS * ROWS_PT

_mesh = plsc.VectorSubcoreMesh(core_axis_name="c", subcore_axis_name="s")


@functools.partial(
    pl.kernel,
    mesh=_mesh,
    out_type=[
        jax.ShapeDtypeStruct((N_PAD,), jnp.float32),
        jax.ShapeDtypeStruct((N_PAD,), jnp.float32),
    ],
    scratch_types=[
        pltpu.VMEM((CPT, CHUNK), jnp.int32),
        pltpu.VMEM((CPT, CHUNK), jnp.float32),
        pltpu.VMEM((ROWS_PT,), jnp.float32),
        pltpu.VMEM_SHARED((N_PAD,), jnp.float32),
    ],
)
def _deg_kernel(dst0_hbm, dst1_hbm, deg0_hbm, deg1_hbm, dstv, onesv, degv,
                acc):
    c = lax.axis_index("c")
    s = lax.axis_index("s")

    def orow(i, carry):
        for k in range(CHUNK // 16):
            onesv[i, pl.ds(k * 16, 16)] = jnp.ones((16,), jnp.float32)
        return carry

    lax.fori_loop(0, CPT, orow, 0)
    base = s * ROWS_PT
    for k in range(ROWS_PT // CHUNK):
        pltpu.sync_copy(onesv.at[0], acc.at[pl.ds(base + k * CHUNK, CHUNK)])
    rem = ROWS_PT - (ROWS_PT // CHUNK) * CHUNK
    if rem:
        pltpu.sync_copy(
            onesv.at[0, pl.ds(0, rem)],
            acc.at[pl.ds(base + (ROWS_PT // CHUNK) * CHUNK, rem)],
        )
    plsc.subcore_barrier()

    def _histogram(dst_hbm):
        pltpu.sync_copy(dst_hbm.at[s], dstv)

        def body(j, carry):
            pltpu.sync_copy(onesv.at[j], acc.at[dstv.at[j]], add=True)
            return carry

        lax.fori_loop(0, CPT, body, 0)

    @pl.when(c == 0)
    def _():
        _histogram(dst0_hbm)

    @pl.when(c == 1)
    def _():
        _histogram(dst1_hbm)

    plsc.subcore_barrier()

    pltpu.sync_copy(acc.at[pl.ds(base, ROWS_PT)], degv)

    @pl.when(c == 0)
    def _():
        pltpu.sync_copy(degv, deg0_hbm.at[pl.ds(base, ROWS_PT)])

    @pl.when(c == 1)
    def _():
        pltpu.sync_copy(degv, deg1_hbm.at[pl.ds(base, ROWS_PT)])


@functools.partial(
    pl.kernel,
    mesh=_mesh,
    out_type=[
        jax.ShapeDtypeStruct((N_PAD, H), jnp.float32),
        jax.ShapeDtypeStruct((N_PAD, H), jnp.float32),
    ],
    scratch_types=[
        pltpu.VMEM((GC, CHUNK), jnp.int32),
        pltpu.VMEM((GC, CHUNK), jnp.int32),
        pltpu.VMEM((CHUNK, H), jnp.float32),
        pltpu.VMEM((CHUNK, H), jnp.float32),
        pltpu.VMEM_SHARED((N_PAD, H), jnp.float32),
        pltpu.SemaphoreType.DMA,
        pltpu.SemaphoreType.DMA,
        pltpu.SemaphoreType.DMA,
        pltpu.SemaphoreType.DMA,
    ],
)
def _scatter_kernel(hlo_hbm, hhi_hbm, src_hbm, dst_hbm, outlo_hbm, outhi_hbm,
                    srcv, dstv, buf0, buf1, acc, sg0, sg1, ss0, ss1):
    c = lax.axis_index("c")
    s = lax.axis_index("s")
    base = s * ROWS_PT

    def zrow(i, carry):
        for k in range(H // 16):
            buf0[i, pl.ds(k * 16, 16)] = jnp.zeros((16,), jnp.float32)
        return carry

    pltpu.async_copy(src_hbm.at[s, pl.ds(0, GC)], srcv, sg0)
    pltpu.async_copy(dst_hbm.at[s, pl.ds(0, GC)], dstv, sg1)
    lax.fori_loop(0, CHUNK, zrow, 0)
    for k in range(ROWS_PT // CHUNK):
        pltpu.sync_copy(buf0, acc.at[pl.ds(base + k * CHUNK, CHUNK)])
    rem = ROWS_PT - (ROWS_PT // CHUNK) * CHUNK
    if rem:
        pltpu.sync_copy(
            buf0.at[pl.ds(0, rem)],
            acc.at[pl.ds(base + (ROWS_PT // CHUNK) * CHUNK, rem)],
        )
    pltpu.make_async_copy(src_hbm.at[s, pl.ds(0, GC)], srcv, sg0).wait()
    pltpu.make_async_copy(dst_hbm.at[s, pl.ds(0, GC)], dstv, sg1).wait()
    plsc.subcore_barrier()

    def run(h_ref):
        def gather(j, buf, sem):
            return pltpu.async_copy(h_ref.at[srcv.at[j]], buf, sem)

        def scat(j, buf, sem):
            return pltpu.async_copy(buf, acc.at[dstv.at[j]], sem, add=True)

        def wait_g(j, buf, sem):
            pltpu.make_async_copy(h_ref.at[srcv.at[j]], buf, sem).wait()

        def wait_s(j, buf, sem):
            pltpu.make_async_copy(buf, acc.at[dstv.at[j]], sem).wait()

        def group(g, carry):
            @pl.when(g > 0)
            def _():
                pltpu.sync_copy(src_hbm.at[s, pl.ds(g * GC, GC)], srcv)
                pltpu.sync_copy(dst_hbm.at[s, pl.ds(g * GC, GC)], dstv)

            gather(0, buf0, sg0)

            def body(i, carry2):
                j0 = 2 * i
                gather(j0 + 1, buf1, sg1)
                wait_g(j0, buf0, sg0)
                pltpu.sync_copy(buf0, acc.at[dstv.at[j0]], add=True)

                @pl.when(j0 + 2 < GC)
                def _():
                    gather(j0 + 2, buf0, sg0)

                wait_g(j0 + 1, buf1, sg1)
                pltpu.sync_copy(buf1, acc.at[dstv.at[j0 + 1]], add=True)
                return carry2

            lax.fori_loop(0, GC // 2, body, 0)
            return carry

        lax.fori_loop(0, CPT // GC, group, 0)

    @pl.when(c == 0)
    def _():
        run(hlo_hbm)

    @pl.when(c == 1)
    def _():
        run(hhi_hbm)

    plsc.subcore_barrier()

    def drain(out_hbm):
        hops = [(k * CHUNK, CHUNK) for k in range(ROWS_PT // CHUNK)]
        rem2 = ROWS_PT - (ROWS_PT // CHUNK) * CHUNK
        if rem2:
            hops.append(((ROWS_PT // CHUNK) * CHUNK, rem2))
        pend = [None, None]
        for k, (off, sz) in enumerate(hops):
            buf = (buf0 if k % 2 == 0 else buf1).at[pl.ds(0, sz)]
            sem = sg0 if k % 2 == 0 else sg1
            if pend[k % 2] is not None:
                pend[k % 2].wait()
            pltpu.sync_copy(acc.at[pl.ds(base + off, sz)], buf)
            pltpu.async_copy(buf, out_hbm.at[pl.ds(base + off, sz)], sem)
            pend[k % 2] = pltpu.make_async_copy(
                buf, out_hbm.at[pl.ds(base + off, sz)], sem)
        for p in pend:
            if p is not None:
                p.wait()

    @pl.when(c == 0)
    def _():
        drain(outlo_hbm)

    @pl.when(c == 1)
    def _():
        drain(outhi_hbm)


_RB = 2000
_GRID = N // _RB


def _b0_body(x_ref, w_ref, deg_ref, lo_ref, hi_ref):
    a = lax.rsqrt(deg_ref[...])
    h = jnp.dot(x_ref[...], w_ref[...], preferred_element_type=jnp.float32) * a
    lo_ref[...] = h[:, :H]
    hi_ref[...] = h[:, H:]


def _mm_scale(x, w, deg):
    return pl.pallas_call(
        _b0_body,
        grid=(_GRID,),
        in_specs=[
            pl.BlockSpec((_RB, D), lambda i: (i, 0)),
            pl.BlockSpec((D, D), lambda i: (0, 0)),
            pl.BlockSpec((_RB, 1), lambda i: (i, 0)),
        ],
        out_specs=[
            pl.BlockSpec((_RB, H), lambda i: (i, 0)),
            pl.BlockSpec((_RB, H), lambda i: (i, 0)),
        ],
        out_shape=[
            jax.ShapeDtypeStruct((N, H), jnp.float32),
            jax.ShapeDtypeStruct((N, H), jnp.float32),
        ],
    )(x, w, deg)


def _b1_body(slo_ref, shi_ref, hlo_ref, hhi_ref, deg0_ref, b_ref, w_ref,
             deg1_ref, lo_ref, hi_ref):
    a0 = lax.rsqrt(deg0_ref[...])
    sfull = jnp.concatenate([slo_ref[...], shi_ref[...]], axis=1)
    hfull = jnp.concatenate([hlo_ref[...], hhi_ref[...]], axis=1)
    z = jnp.maximum(a0 * (sfull + hfull) + b_ref[...], 0.0)
    h1 = jnp.dot(z, w_ref[...], preferred_element_type=jnp.float32)
    h1 = h1 * lax.rsqrt(deg1_ref[...])
    lo_ref[...] = h1[:, :H]
    hi_ref[...] = h1[:, H:]


def _fused_mid(slo, shi, hlo, hhi, deg0, b0, w1, deg1):
    return pl.pallas_call(
        _b1_body,
        grid=(_GRID,),
        in_specs=[
            pl.BlockSpec((_RB, H), lambda i: (i, 0)),
            pl.BlockSpec((_RB, H), lambda i: (i, 0)),
            pl.BlockSpec((_RB, H), lambda i: (i, 0)),
            pl.BlockSpec((_RB, H), lambda i: (i, 0)),
            pl.BlockSpec((_RB, 1), lambda i: (i, 0)),
            pl.BlockSpec((1, D), lambda i: (0, 0)),
            pl.BlockSpec((D, D), lambda i: (0, 0)),
            pl.BlockSpec((_RB, 1), lambda i: (i, 0)),
        ],
        out_specs=[
            pl.BlockSpec((_RB, H), lambda i: (i, 0)),
            pl.BlockSpec((_RB, H), lambda i: (i, 0)),
        ],
        out_shape=[
            jax.ShapeDtypeStruct((N, H), jnp.float32),
            jax.ShapeDtypeStruct((N, H), jnp.float32),
        ],
    )(slo, shi, hlo, hhi, deg0, b0, w1, deg1)


def _b2_body(slo_ref, shi_ref, hlo_ref, hhi_ref, deg_ref, b_ref, out_ref):
    a = lax.rsqrt(deg_ref[...])
    sfull = jnp.concatenate([slo_ref[...], shi_ref[...]], axis=1)
    hfull = jnp.concatenate([hlo_ref[...], hhi_ref[...]], axis=1)
    out_ref[...] = a * (sfull + hfull) + b_ref[...]


def _final(slo, shi, hlo, hhi, deg, b):
    return pl.pallas_call(
        _b2_body,
        grid=(_GRID,),
        in_specs=[
            pl.BlockSpec((_RB, H), lambda i: (i, 0)),
            pl.BlockSpec((_RB, H), lambda i: (i, 0)),
            pl.BlockSpec((_RB, H), lambda i: (i, 0)),
            pl.BlockSpec((_RB, H), lambda i: (i, 0)),
            pl.BlockSpec((_RB, 1), lambda i: (i, 0)),
            pl.BlockSpec((1, D), lambda i: (0, 0)),
        ],
        out_specs=pl.BlockSpec((_RB, D), lambda i: (i, 0)),
        out_shape=jax.ShapeDtypeStruct((N, D), jnp.float32),
    )(slo, shi, hlo, hhi, deg, b)


def _prep_edges(src, dst):
    pad_src = (jnp.arange(NS * PAD_PER_TILE, dtype=jnp.int32) * 37) % N
    pad_dst = jnp.full((NS * PAD_PER_TILE,), N, jnp.int32)
    src_t = jnp.concatenate(
        [src.reshape(NS, E_PER_TILE_REAL), pad_src.reshape(NS, PAD_PER_TILE)],
        axis=1).reshape(NS, CPT, CHUNK)
    dst_t = jnp.concatenate(
        [dst.reshape(NS, E_PER_TILE_REAL), pad_dst.reshape(NS, PAD_PER_TILE)],
        axis=1).reshape(NS, CPT, CHUNK)
    return src_t, dst_t


def kernel(x, edge_index_list, W0, b0, W1, b1):
    e = edge_index_list.astype(jnp.int32)
    src0, dst0 = _prep_edges(e[0, 0], e[0, 1])
    src1, dst1 = _prep_edges(e[1, 0], e[1, 1])

    deg0, deg1 = _deg_kernel(dst0, dst1)
    deg0 = deg0.reshape(N_PAD, 1)
    deg1 = deg1.reshape(N_PAD, 1)

    h0lo, h0hi = _mm_scale(x, W0, deg0)
    s0lo, s0hi = _scatter_kernel(h0lo, h0hi, src0, dst0)
    h1lo, h1hi = _fused_mid(s0lo, s0hi, h0lo, h0hi, deg0,
                            b0.reshape(1, D), W1, deg1)
    s1lo, s1hi = _scatter_kernel(h1lo, h1hi, src1, dst1)
    return _final(s1lo, s1hi, h1lo, h1hi, deg1, b1.reshape(1, D))
